# baseline (speedup 1.0000x reference)
import dataclasses
import functools

import jax
import jax.numpy as jnp
from jax import lax
from jax.experimental import pallas as pl
from jax.experimental.pallas import tpu as pltpu
from jax.experimental.pallas import tpu_sc as plsc

B = 16384
LROW = 200
NCORES = 2
NSUB = 16
NW = NCORES * NSUB
ROWS_PER_W = B // NW
ELEMS_PER_W = ROWS_PER_W * LROW
PAIRS = ROWS_PER_W // 2
LANES = 16
VECS_PER_PAIR = 2 * LROW // LANES
CHUNK = 128
NCHUNK = ROWS_PER_W // CHUNK


def _sc_rowsum(a2d, table128):
    mesh = plsc.VectorSubcoreMesh(core_axis_name="c", subcore_axis_name="s")
    cp = pltpu.CompilerParams()
    if "needs_layout_passes" in pltpu.CompilerParams.__dataclass_fields__:
        cp = dataclasses.replace(cp, needs_layout_passes=False)
    if "use_tc_tiling_on_sc" in pltpu.CompilerParams.__dataclass_fields__:
        cp = dataclasses.replace(cp, use_tc_tiling_on_sc=False)

    @functools.partial(
        pl.kernel,
        compiler_params=cp,
        out_type=jax.ShapeDtypeStruct((B,), jnp.float32),
        mesh=mesh,
        scratch_types=[
            pltpu.VMEM((128,), jnp.float32),
            pltpu.VMEM((CHUNK, LROW), jnp.int32),
            pltpu.VMEM((CHUNK, LROW), jnp.int32),
            pltpu.VMEM((ROWS_PER_W,), jnp.float32),
            pltpu.SemaphoreType.DMA,
            pltpu.SemaphoreType.DMA,
        ],
    )
    def k(a_hbm, table_hbm, out_hbm, table_v, b0, b1, out_v, s0, s1):
        wid = lax.axis_index("s") * NCORES + lax.axis_index("c")
        row0 = wid * ROWS_PER_W
        pltpu.sync_copy(table_hbm, table_v)

        lane = lax.iota(jnp.int32, LANES)
        mask_hi = lane >= (LANES - LROW % LANES)

        bufs = (b0, b1)
        sems = (s0, s1)
        cps = [None, None]
        cps[0] = pltpu.async_copy(a_hbm.at[pl.ds(row0, CHUNK)], b0, s0)
        for c in range(NCHUNK):
            cur = c & 1
            nxt = cur ^ 1
            if c + 1 < NCHUNK:
                cps[nxt] = pltpu.async_copy(
                    a_hbm.at[pl.ds(row0 + (c + 1) * CHUNK, CHUNK)],
                    bufs[nxt],
                    sems[nxt],
                )
            cps[cur].wait()
            block_v = bufs[cur]

            @pl.loop(0, CHUNK // LANES)
            def _(g):
                acc_out = jnp.zeros((LANES,), jnp.float32)
                for jr in range(LANES):
                    r = g * LANES + jr
                    acc = jnp.zeros((LANES,), jnp.float32)
                    for kv in range(LROW // LANES):
                        idx = block_v[r, pl.ds(kv * LANES, LANES)]
                        acc = acc + plsc.load_gather(table_v, [idx])
                    idx = block_v[r, pl.ds(LROW - LANES, LANES)]
                    v = plsc.load_gather(table_v, [idx])
                    acc = acc + jnp.where(mask_hi, v, 0.0)
                    acc_out = jnp.where(lane == jr, jnp.sum(acc), acc_out)
                out_v[pl.ds(c * CHUNK + g * LANES, LANES)] = acc_out

        pltpu.sync_copy(out_v, out_hbm.at[pl.ds(wid * ROWS_PER_W, ROWS_PER_W)])

    return k(a2d, table128)


@jax.jit
def kernel(atomic_numbers, ref_energy_weight):
    table128 = jnp.zeros((128,), jnp.float32).at[:100].set(
        ref_energy_weight[:, 0]
    )
    return _sc_rowsum(atomic_numbers, table128)


# speedup vs baseline: 2.6916x; 2.6916x over previous
import dataclasses
import functools

import jax
import jax.numpy as jnp
from jax import lax
from jax.experimental import pallas as pl
from jax.experimental.pallas import tpu as pltpu
from jax.experimental.pallas import tpu_sc as plsc

B = 16384
LROW = 200
NCORES = 2
NSUB = 16
NW = NCORES * NSUB
NB = B // NW
LANES = 16
SUB = 128


def _sc_rowsum(xt, table2):
    mesh = plsc.VectorSubcoreMesh(core_axis_name="c", subcore_axis_name="s")
    cp = pltpu.CompilerParams()
    if "needs_layout_passes" in pltpu.CompilerParams.__dataclass_fields__:
        cp = dataclasses.replace(cp, needs_layout_passes=False)
    if "use_tc_tiling_on_sc" in pltpu.CompilerParams.__dataclass_fields__:
        cp = dataclasses.replace(cp, use_tc_tiling_on_sc=True)

    @functools.partial(
        pl.kernel,
        out_type=jax.ShapeDtypeStruct((B,), jnp.float32),
        mesh=mesh,
        compiler_params=cp,
        scratch_types=[
            pltpu.VMEM((128 * 128,), jnp.float32),
            pltpu.VMEM((LROW, SUB), jnp.int32),
            pltpu.VMEM((LROW, SUB), jnp.int32),
            pltpu.VMEM((NB,), jnp.float32),
            pltpu.SemaphoreType.DMA,
            pltpu.SemaphoreType.DMA,
        ],
    )
    def k(xt_hbm, table_hbm, out_hbm, table_v, b0, b1, out_v, s0, s1):
        wid = lax.axis_index("s") * NCORES + lax.axis_index("c")
        col0 = wid * NB
        pltpu.sync_copy(table_hbm, table_v)

        bufs = (b0, b1)
        sems = (s0, s1)
        cps = [None, None]
        cps[0] = pltpu.async_copy(xt_hbm.at[:, pl.ds(col0, SUB)], b0, s0)
        for sb in range(NB // SUB):
            cur = sb & 1
            nxt = cur ^ 1
            if sb + 1 < NB // SUB:
                cps[nxt] = pltpu.async_copy(
                    xt_hbm.at[:, pl.ds(col0 + (sb + 1) * SUB, SUB)],
                    bufs[nxt],
                    sems[nxt],
                )
            cps[cur].wait()
            block_v = bufs[cur]
            zero = jnp.zeros((LANES,), jnp.float32)

            @pl.loop(
                0, LROW // 2, init_carry=(zero,) * (SUB // LANES), unroll=2
            )
            def accs(lp, carry):
                l = lp * 2
                new = []
                for j in range(SUB // LANES):
                    lo = block_v[l, pl.ds(j * LANES, LANES)]
                    hi = block_v[l + 1, pl.ds(j * LANES, LANES)]
                    comb = lo + (hi << 7)
                    new.append(carry[j] + plsc.load_gather(table_v, [comb]))
                return tuple(new)

            for j in range(SUB // LANES):
                out_v[pl.ds(sb * SUB + j * LANES, LANES)] = accs[j]

        pltpu.sync_copy(out_v, out_hbm.at[pl.ds(col0, NB)])

    return k(xt, table2)


@jax.jit
def kernel(atomic_numbers, ref_energy_weight):
    w = ref_energy_weight.reshape(100)
    w128 = jnp.zeros((128,), jnp.float32).at[:100].set(w)
    table2 = (w128[None, :] + w128[:, None]).reshape(-1)
    return _sc_rowsum(atomic_numbers.T, table2)
